# baseline (speedup 1.0000x reference)
import functools
import jax
import jax.numpy as jnp
import numpy as np
from jax import lax
from jax.experimental import pallas as pl
from jax.experimental.pallas import tpu as pltpu
from jax.experimental.pallas import tpu_sc as plsc

CFG_SCALE = 2.0
TEMPERATURE = 0.85
TOP_P = 0.9
AUDIO_START_ID = 151669
AUDIO_END_ID = 215669
EOS_TOKEN_ID = 151645
SAMPLE_SEED = 1
B = 8
V = 1000000

W0 = 151552
WW = 65536
EOS_I = EOS_TOKEN_ID - W0
A0_I = AUDIO_START_ID - W0
A1_I = AUDIO_END_ID - W0
N_CHUNK = 16
BAND_C2 = 45056
OFF_C2 = 20480

CH = 16384
NV = WW // 16
NVC = CH // 16
NEG = float(np.finfo(np.float32).min)


def _lane_reduce(vec, op):
    acc = vec[0]
    for j in range(1, 16):
        acc = op(acc, vec[j])
    return acc


def _hist_threshold_round(arr, hist, sums, rem, shift, width_mask, prefix_shift,
                          prefix_val, use_prefix):
    f32 = jnp.float32

    def zero_body(i, _):
        hist[pl.ds(i * 16, 16)] = jnp.zeros((16,), f32)
        return 0

    lax.fori_loop(0, 256, zero_body, 0)

    def scat_body(i, _):
        v = arr[pl.ds(i * 16, 16)]
        bits = lax.bitcast_convert_type(v, jnp.int32)
        idx = lax.shift_right_logical(bits, np.int32(shift)) & np.int32(width_mask)
        if use_prefix:
            msk = lax.shift_right_logical(bits, np.int32(prefix_shift)) == prefix_val
        else:
            msk = bits >= 0
        plsc.addupdate_scatter(hist, [idx], v, mask=msk)
        return 0

    lax.fori_loop(0, NV, scat_body, 0)

    def suma_body(i, _):
        sums[i] = plsc.cumsum(hist[pl.ds(i * 16, 16)])[15]
        return 0

    lax.fori_loop(0, 256, suma_body, 0)

    def walk_body(i, carry):
        sel, above, run = carry
        j = np.int32(255) - i
        s = sums[j]
        run2 = run + s
        cross = (run2 > rem) & (sel < 0)
        sel = jnp.where(cross, j, sel)
        above = jnp.where(cross, run, above)
        return sel, above, run2

    sel, above, _ = lax.fori_loop(
        0, 256, walk_body,
        (jnp.int32(-1), f32(0.0), f32(0.0)))
    sel = jnp.maximum(sel, 0)
    rem2 = rem - above

    v = hist[pl.ds(sel * 16, 16)]
    rv = lax.rev(v, (0,))
    cs = plsc.cumsum(rv)
    cond = cs > rem2
    cnt = _lane_reduce(jnp.where(cond, jnp.int32(1), jnp.int32(0)),
                       lambda a, b: a + b)
    cnt = jnp.maximum(cnt, 1)
    binv = sel * 16 + (cnt - 1)
    above2 = _lane_reduce(jnp.where(cond, f32(0.0), rv), lambda a, b: a + b)
    return binv, rem2 - above2


def _sc_body(cond_hbm, uncond_hbm, bprobs_hbm, cfg2_hbm, arr, cw, uw, hist, sums):
    f32 = jnp.float32
    wid = lax.axis_index("s") * 2 + lax.axis_index("c")

    @pl.when(wid < B)
    def _row():
        row = wid
        iota = lax.iota(jnp.int32, 16)

        def p1_chunk(k, mx):
            pltpu.sync_copy(cond_hbm.at[row, pl.ds(k * CH, CH)], cw)
            pltpu.sync_copy(uncond_hbm.at[row, pl.ds(k * CH, CH)], uw)

            def body(i, mxi):
                cvec = cw[pl.ds(i * 16, 16)]
                uvec = uw[pl.ds(i * 16, 16)]
                cfg = uvec + f32(CFG_SCALE) * (cvec - uvec)
                col = iota + (k * CH + i * 16)
                act = (col == EOS_I) | ((col >= A0_I) & (col < A1_I))
                cfgm = jnp.where(act, cfg, f32(NEG))
                arr[pl.ds(k * CH + i * 16, 16)] = cfgm
                return jnp.maximum(mxi, cfgm)

            return lax.fori_loop(0, NVC, body, mx)

        mxv = lax.fori_loop(0, 4, p1_chunk, jnp.full((16,), NEG, f32))
        m = _lane_reduce(mxv, jnp.maximum)

        def p2_body(i, z):
            v = arr[pl.ds(i * 16, 16)]
            pv = jnp.exp(v - m)
            arr[pl.ds(i * 16, 16)] = pv
            return z + pv

        zv = lax.fori_loop(0, NV, p2_body, jnp.zeros((16,), f32))
        rem = f32(TOP_P) * _lane_reduce(zv, lambda a, b: a + b)

        b1, rem = _hist_threshold_round(arr, hist, sums, rem, 18, 0xFFF, 0, 0, False)
        b2, rem = _hist_threshold_round(arr, hist, sums, rem, 6, 0xFFF, 18, b1, True)
        pre18 = lax.shift_left(b1, np.int32(12)) | b2
        b3, rem = _hist_threshold_round(arr, hist, sums, rem, 0, 0x3F, 6, pre18, True)
        thr = (lax.shift_left(b1, np.int32(18))
               | lax.shift_left(b2, np.int32(6)) | b3)

        inv_t = f32(1.0 / TEMPERATURE)

        def p4_chunk(k, z2):
            pltpu.sync_copy(cond_hbm.at[row, pl.ds(k * CH, CH)], cw)
            pltpu.sync_copy(uncond_hbm.at[row, pl.ds(k * CH, CH)], uw)

            def body(i, z2i):
                cvec = cw[pl.ds(i * 16, 16)]
                uvec = uw[pl.ds(i * 16, 16)]
                cfg = uvec + f32(CFG_SCALE) * (cvec - uvec)
                col = iota + (k * CH + i * 16)
                act = (col == EOS_I) | ((col >= A0_I) & (col < A1_I))
                cfg = jnp.where(act, cfg, f32(NEG))
                pv = arr[pl.ds(k * CH + i * 16, 16)]
                keep = lax.bitcast_convert_type(pv, jnp.int32) >= thr
                cfg2 = jnp.where(keep, cfg / f32(TEMPERATURE), -jnp.inf)
                cw[pl.ds(i * 16, 16)] = cfg2
                p2 = jnp.where(keep, jnp.exp((cfg - m) * inv_t), f32(0.0))
                arr[pl.ds(k * CH + i * 16, 16)] = p2
                return z2i + p2

            z2 = lax.fori_loop(0, NVC, body, z2)
            pltpu.sync_copy(cw, cfg2_hbm.at[row, pl.ds(k * CH, CH)])
            return z2

        z2v = lax.fori_loop(0, 4, p4_chunk, jnp.zeros((16,), f32))
        z2 = _lane_reduce(z2v, lambda a, b: a + b)

        def p5_body(i, _):
            arr[pl.ds(i * 16, 16)] = arr[pl.ds(i * 16, 16)] / z2
            return 0

        lax.fori_loop(0, NV, p5_body, 0)

        def p5_out(k, _):
            pltpu.sync_copy(arr.at[pl.ds(k * CH, CH)],
                            bprobs_hbm.at[row, pl.ds(k * CH, CH)])
            return 0

        lax.fori_loop(0, 4, p5_out, 0)


def _sc_topp(cond_w, uncond_w):
    mesh = plsc.VectorSubcoreMesh(core_axis_name="c", subcore_axis_name="s",
                                  num_cores=2, num_subcores=16)
    fn = pl.kernel(
        _sc_body,
        compiler_params=pltpu.CompilerParams(needs_layout_passes=False),
        out_type=[
            jax.ShapeDtypeStruct((B, WW), jnp.float32),
            jax.ShapeDtypeStruct((B, WW), jnp.float32),
        ],
        mesh=mesh,
        scratch_types=[
            pltpu.VMEM((WW,), jnp.float32),
            pltpu.VMEM((CH,), jnp.float32),
            pltpu.VMEM((CH,), jnp.float32),
            pltpu.VMEM((4096,), jnp.float32),
            pltpu.SMEM((256,), jnp.float32),
        ],
    )
    return fn(cond_w, uncond_w)


def _rotl(x, d):
    return lax.shift_left(x, np.int32(d)) | lax.shift_right_logical(x, np.int32(32 - d))


def _threefry(x0, x1):
    ks0 = np.int32(0)
    ks1 = np.int32(SAMPLE_SEED)
    ks2 = np.int32(ks0 ^ ks1 ^ np.int32(0x1BD11BDA))
    rot1 = (13, 15, 26, 6)
    rot2 = (17, 29, 16, 24)
    x0 = x0 + ks0
    x1 = x1 + ks1
    for r in rot1:
        x0 = x0 + x1; x1 = _rotl(x1, r); x1 = x0 ^ x1
    x0 = x0 + ks1; x1 = x1 + ks2 + np.int32(1)
    for r in rot2:
        x0 = x0 + x1; x1 = _rotl(x1, r); x1 = x0 ^ x1
    x0 = x0 + ks2; x1 = x1 + ks0 + np.int32(2)
    for r in rot1:
        x0 = x0 + x1; x1 = _rotl(x1, r); x1 = x0 ^ x1
    x0 = x0 + ks0; x1 = x1 + ks1 + np.int32(3)
    for r in rot2:
        x0 = x0 + x1; x1 = _rotl(x1, r); x1 = x0 ^ x1
    x0 = x0 + ks1; x1 = x1 + ks2 + np.int32(4)
    for r in rot1:
        x0 = x0 + x1; x1 = _rotl(x1, r); x1 = x0 ^ x1
    x0 = x0 + ks2; x1 = x1 + ks0 + np.int32(5)
    return x0, x1


def _gumbel_window():
    row = lax.broadcasted_iota(jnp.int32, (B, WW), 0)
    col = lax.broadcasted_iota(jnp.int32, (B, WW), 1)
    flat = row * np.int32(V) + (col + np.int32(W0))
    o1, o2 = _threefry(jnp.zeros((B, WW), jnp.int32), flat)
    bits = o1 ^ o2
    fb = lax.shift_right_logical(bits, np.int32(9)) | np.int32(0x3F800000)
    f = lax.bitcast_convert_type(fb, jnp.float32) - jnp.float32(1.0)
    tiny = jnp.float32(np.finfo(np.float32).tiny)
    u = jnp.maximum(tiny, f * (jnp.float32(1.0) - tiny) + tiny)
    return -jnp.log(-jnp.log(u))


def _tc_kernel(bprobs_ref, cfg2_ref, probs_ref, ntok_ref):
    c = pl.program_id(0)

    @pl.when(c == 0)
    def _sample():
        col = lax.broadcasted_iota(jnp.int32, (B, WW), 1)
        g = _gumbel_window()
        score = cfg2_ref[...] + g
        smax = jnp.max(score, axis=1, keepdims=True)
        win = jnp.where(score == smax, col, np.int32(2 * WW))
        idx = jnp.min(win, axis=1, keepdims=True) + np.int32(W0)
        ntok_ref[...] = jnp.broadcast_to(idx, (B, 128))

    probs_ref[...] = jnp.zeros((B, WW), jnp.float32)

    @pl.when(c == 2)
    def _band_lo():
        probs_ref[:, OFF_C2:] = bprobs_ref[:, :BAND_C2]

    @pl.when(c == 3)
    def _band_hi():
        probs_ref[:, :OFF_C2] = bprobs_ref[:, BAND_C2:]


def kernel(cond_logits, uncond_logits):
    cond_w = lax.slice(cond_logits, (0, W0), (B, W0 + WW))
    uncond_w = lax.slice(uncond_logits, (0, W0), (B, W0 + WW))
    bprobs, cfg2 = _sc_topp(cond_w, uncond_w)
    probs, ntok = pl.pallas_call(
        _tc_kernel,
        grid=(N_CHUNK,),
        in_specs=[
            pl.BlockSpec((B, WW), lambda c: (0, 0)),
            pl.BlockSpec((B, WW), lambda c: (0, 0)),
        ],
        out_specs=[
            pl.BlockSpec((B, WW), lambda c: (0, c)),
            pl.BlockSpec((B, 128), lambda c: (0, 0)),
        ],
        out_shape=[
            jax.ShapeDtypeStruct((B, V), jnp.float32),
            jax.ShapeDtypeStruct((B, 128), jnp.int32),
        ],
    )(bprobs, cfg2)
    return probs, ntok[:, 0]


# speedup vs baseline: 1.3159x; 1.3159x over previous
import functools
import jax
import jax.numpy as jnp
import numpy as np
from jax import lax
from jax.experimental import pallas as pl
from jax.experimental.pallas import tpu as pltpu
from jax.experimental.pallas import tpu_sc as plsc

CFG_SCALE = 2.0
TEMPERATURE = 0.85
TOP_P = 0.9
AUDIO_START_ID = 151669
AUDIO_END_ID = 215669
EOS_TOKEN_ID = 151645
SAMPLE_SEED = 1
B = 8
V = 1000000

W0 = 151552
WW = 65536
EOS_I = EOS_TOKEN_ID - W0
A0_I = AUDIO_START_ID - W0
A1_I = AUDIO_END_ID - W0
N_CHUNK = 16
BAND_C2 = 45056
OFF_C2 = 20480

CH = 16384
NV = WW // 16
NVC = CH // 16
UNROLL = 8
NEG = float(np.finfo(np.float32).min)


def _lane_reduce(vec, op):
    acc = vec[0]
    for j in range(1, 16):
        acc = op(acc, vec[j])
    return acc


def _ukey(z):
    bits = lax.bitcast_convert_type(z, jnp.int32) | np.int32(-2147483648)
    return bits ^ np.int32(-1)


def _walk(hist, sums, rem, nvreg):
    f32 = jnp.float32

    def suma_body(i, _):
        sums[i] = plsc.cumsum(hist[pl.ds(i * 16, 16)])[15]
        return 0

    lax.fori_loop(0, nvreg, suma_body, 0)

    def walk_body(i, carry):
        sel, above, run = carry
        j = np.int32(nvreg - 1) - i
        run2 = run + sums[j]
        cross = (run2 > rem) & (sel < 0)
        sel = jnp.where(cross, j, sel)
        above = jnp.where(cross, run, above)
        return sel, above, run2

    sel, above, _ = lax.fori_loop(0, nvreg, walk_body,
                                  (jnp.int32(-1), f32(0.0), f32(0.0)))
    sel = jnp.maximum(sel, 0)
    rem2 = rem - above

    v = hist[pl.ds(sel * 16, 16)]
    rv = lax.rev(v, (0,))
    cs = plsc.cumsum(rv)
    cond = cs > rem2
    cnt = _lane_reduce(jnp.where(cond, jnp.int32(1), jnp.int32(0)),
                       lambda a, b: a + b)
    cnt = jnp.maximum(cnt, 1)
    binv = sel * 16 + (cnt - 1)
    above2 = _lane_reduce(jnp.where(cond, f32(0.0), rv), lambda a, b: a + b)
    return binv, rem2 - above2


def _zero_hist(hist, nvreg):
    def body(i, _):
        hist[pl.ds(i * 16, 16)] = jnp.zeros((16,), jnp.float32)
        return 0
    lax.fori_loop(0, nvreg, body, 0)


def _sc_body(cond_hbm, uncond_hbm, bprobs_hbm, cfg2_hbm, arr, cw, uw, ob, hist,
             sums):
    f32 = jnp.float32
    wid = lax.axis_index("s") * 2 + lax.axis_index("c")

    @pl.when(wid < B)
    def _row():
        row = wid
        iota = lax.iota(jnp.int32, 16)
        inv_t = f32(1.0 / TEMPERATURE)

        def pa_chunk(k, mx):
            pltpu.sync_copy(cond_hbm.at[row, pl.ds(k * CH, CH)], cw)
            pltpu.sync_copy(uncond_hbm.at[row, pl.ds(k * CH, CH)], uw)

            def body(i, mxi):
                for j in range(UNROLL):
                    off = i * (16 * UNROLL) + j * 16
                    cvec = cw[pl.ds(off, 16)]
                    uvec = uw[pl.ds(off, 16)]
                    cfg = uvec + f32(CFG_SCALE) * (cvec - uvec)
                    ob[pl.ds(off, 16)] = cfg / f32(TEMPERATURE)
                    col = iota + (k * CH + off)
                    act = (col == EOS_I) | ((col >= A0_I) & (col < A1_I))
                    cfgm = jnp.where(act, cfg, f32(NEG))
                    arr[pl.ds(k * CH + off, 16)] = cfgm
                    mxi = jnp.maximum(mxi, cfgm)
                return mxi

            mx = lax.fori_loop(0, NVC // UNROLL, body, mx)
            pltpu.sync_copy(ob, cfg2_hbm.at[row, pl.ds(k * CH, CH)])
            return mx

        mxv = lax.fori_loop(0, 4, pa_chunk, jnp.full((16,), NEG, f32))
        m = _lane_reduce(mxv, jnp.maximum)

        _zero_hist(hist, 64)

        def pb_body(i, z):
            for j in range(UNROLL):
                off = i * (16 * UNROLL) + j * 16
                zv = arr[pl.ds(off, 16)] - m
                arr[pl.ds(off, 16)] = zv
                pv = jnp.exp(zv)
                z = z + pv
                idx = lax.shift_right_logical(_ukey(zv), np.int32(21))
                plsc.addupdate_scatter(hist, [idx], pv)
            return z

        zv = lax.fori_loop(0, NV // UNROLL, pb_body, jnp.zeros((16,), f32))
        rem = f32(TOP_P) * _lane_reduce(zv, lambda a, b: a + b)
        b1, rem = _walk(hist, sums, rem, 64)

        _zero_hist(hist, 64)

        def r2_body(i, _):
            for j in range(UNROLL):
                off = i * (16 * UNROLL) + j * 16
                uk = _ukey(arr[pl.ds(off, 16)])
                pv = jnp.exp(arr[pl.ds(off, 16)])
                msk = lax.shift_right_logical(uk, np.int32(21)) == b1
                idx = lax.shift_right_logical(uk, np.int32(11)) & np.int32(0x3FF)
                plsc.addupdate_scatter(hist, [idx], pv, mask=msk)
            return 0

        lax.fori_loop(0, NV // UNROLL, r2_body, 0)
        b2, rem = _walk(hist, sums, rem, 64)

        pre = lax.shift_left(b1, np.int32(10)) | b2
        _zero_hist(hist, 128)

        def r3_body(i, _):
            for j in range(UNROLL):
                off = i * (16 * UNROLL) + j * 16
                uk = _ukey(arr[pl.ds(off, 16)])
                pv = jnp.exp(arr[pl.ds(off, 16)])
                msk = lax.shift_right_logical(uk, np.int32(11)) == pre
                idx = uk & np.int32(0x7FF)
                plsc.addupdate_scatter(hist, [idx], pv, mask=msk)
            return 0

        lax.fori_loop(0, NV // UNROLL, r3_body, 0)
        b3, _ = _walk(hist, sums, rem, 128)
        thr = (lax.shift_left(b1, np.int32(21))
               | lax.shift_left(b2, np.int32(11)) | b3)

        def pc_body(i, z2):
            for j in range(UNROLL):
                off = i * (16 * UNROLL) + j * 16
                zv = arr[pl.ds(off, 16)]
                keep = _ukey(zv) >= thr
                p2 = jnp.where(keep, jnp.exp(zv * inv_t), f32(0.0))
                arr[pl.ds(off, 16)] = p2
                z2 = z2 + p2
            return z2

        z2v = lax.fori_loop(0, NV // UNROLL, pc_body, jnp.zeros((16,), f32))
        z2 = _lane_reduce(z2v, lambda a, b: a + b)

        def pd_body(i, _):
            for j in range(UNROLL):
                off = i * (16 * UNROLL) + j * 16
                arr[pl.ds(off, 16)] = arr[pl.ds(off, 16)] / z2
            return 0

        lax.fori_loop(0, NV // UNROLL, pd_body, 0)

        def pd_out(k, _):
            pltpu.sync_copy(arr.at[pl.ds(k * CH, CH)],
                            bprobs_hbm.at[row, pl.ds(k * CH, CH)])
            return 0

        lax.fori_loop(0, 4, pd_out, 0)


def _sc_topp(cond_w, uncond_w):
    mesh = plsc.VectorSubcoreMesh(core_axis_name="c", subcore_axis_name="s",
                                  num_cores=2, num_subcores=16)
    fn = pl.kernel(
        _sc_body,
        compiler_params=pltpu.CompilerParams(needs_layout_passes=False),
        out_type=[
            jax.ShapeDtypeStruct((B, WW), jnp.float32),
            jax.ShapeDtypeStruct((B, WW), jnp.float32),
        ],
        mesh=mesh,
        scratch_types=[
            pltpu.VMEM((WW,), jnp.float32),
            pltpu.VMEM((CH,), jnp.float32),
            pltpu.VMEM((CH,), jnp.float32),
            pltpu.VMEM((CH,), jnp.float32),
            pltpu.VMEM((2048,), jnp.float32),
            pltpu.SMEM((128,), jnp.float32),
        ],
    )
    return fn(cond_w, uncond_w)


def _rotl(x, d):
    return lax.shift_left(x, np.int32(d)) | lax.shift_right_logical(x, np.int32(32 - d))


def _threefry(x0, x1):
    ks0 = np.int32(0)
    ks1 = np.int32(SAMPLE_SEED)
    ks2 = np.int32(ks0 ^ ks1 ^ np.int32(0x1BD11BDA))
    rot1 = (13, 15, 26, 6)
    rot2 = (17, 29, 16, 24)
    x0 = x0 + ks0
    x1 = x1 + ks1
    for r in rot1:
        x0 = x0 + x1; x1 = _rotl(x1, r); x1 = x0 ^ x1
    x0 = x0 + ks1; x1 = x1 + ks2 + np.int32(1)
    for r in rot2:
        x0 = x0 + x1; x1 = _rotl(x1, r); x1 = x0 ^ x1
    x0 = x0 + ks2; x1 = x1 + ks0 + np.int32(2)
    for r in rot1:
        x0 = x0 + x1; x1 = _rotl(x1, r); x1 = x0 ^ x1
    x0 = x0 + ks0; x1 = x1 + ks1 + np.int32(3)
    for r in rot2:
        x0 = x0 + x1; x1 = _rotl(x1, r); x1 = x0 ^ x1
    x0 = x0 + ks1; x1 = x1 + ks2 + np.int32(4)
    for r in rot1:
        x0 = x0 + x1; x1 = _rotl(x1, r); x1 = x0 ^ x1
    x0 = x0 + ks2; x1 = x1 + ks0 + np.int32(5)
    return x0, x1


def _gumbel_window():
    row = lax.broadcasted_iota(jnp.int32, (B, WW), 0)
    col = lax.broadcasted_iota(jnp.int32, (B, WW), 1)
    flat = row * np.int32(V) + (col + np.int32(W0))
    o1, o2 = _threefry(jnp.zeros((B, WW), jnp.int32), flat)
    bits = o1 ^ o2
    fb = lax.shift_right_logical(bits, np.int32(9)) | np.int32(0x3F800000)
    f = lax.bitcast_convert_type(fb, jnp.float32) - jnp.float32(1.0)
    tiny = jnp.float32(np.finfo(np.float32).tiny)
    u = jnp.maximum(tiny, f * (jnp.float32(1.0) - tiny) + tiny)
    return -jnp.log(-jnp.log(u))


def _tc_kernel(bprobs_ref, cfg2_ref, probs_ref, ntok_ref):
    c = pl.program_id(0)

    @pl.when(c == 0)
    def _sample():
        col = lax.broadcasted_iota(jnp.int32, (B, WW), 1)
        g = _gumbel_window()
        keep = bprobs_ref[...] > 0.0
        score = jnp.where(keep, cfg2_ref[...] + g, -jnp.inf)
        smax = jnp.max(score, axis=1, keepdims=True)
        win = jnp.where(score == smax, col, np.int32(2 * WW))
        idx = jnp.min(win, axis=1, keepdims=True) + np.int32(W0)
        ntok_ref[...] = jnp.broadcast_to(idx, (B, 128))

    probs_ref[...] = jnp.zeros((B, WW), jnp.float32)

    @pl.when(c == 2)
    def _band_lo():
        probs_ref[:, OFF_C2:] = bprobs_ref[:, :BAND_C2]

    @pl.when(c == 3)
    def _band_hi():
        probs_ref[:, :OFF_C2] = bprobs_ref[:, BAND_C2:]


def kernel(cond_logits, uncond_logits):
    cond_w = lax.slice(cond_logits, (0, W0), (B, W0 + WW))
    uncond_w = lax.slice(uncond_logits, (0, W0), (B, W0 + WW))
    bprobs, cfg2 = _sc_topp(cond_w, uncond_w)
    probs, ntok = pl.pallas_call(
        _tc_kernel,
        grid=(N_CHUNK,),
        in_specs=[
            pl.BlockSpec((B, WW), lambda c: (0, 0)),
            pl.BlockSpec((B, WW), lambda c: (0, 0)),
        ],
        out_specs=[
            pl.BlockSpec((B, WW), lambda c: (0, c)),
            pl.BlockSpec((B, 128), lambda c: (0, 0)),
        ],
        out_shape=[
            jax.ShapeDtypeStruct((B, V), jnp.float32),
            jax.ShapeDtypeStruct((B, 128), jnp.int32),
        ],
    )(bprobs, cfg2)
    return probs, ntok[:, 0]


# speedup vs baseline: 2.7013x; 2.0528x over previous
import functools
import jax
import jax.numpy as jnp
import numpy as np
from jax import lax
from jax.experimental import pallas as pl
from jax.experimental.pallas import tpu as pltpu
from jax.experimental.pallas import tpu_sc as plsc

CFG_SCALE = 2.0
TEMPERATURE = 0.85
TOP_P = 0.9
AUDIO_START_ID = 151669
AUDIO_END_ID = 215669
EOS_TOKEN_ID = 151645
SAMPLE_SEED = 1
B = 8
V = 1000000

W0 = 151552
WW = 65536
EOS_I = EOS_TOKEN_ID - W0
A0_I = AUDIO_START_ID - W0
A1_I = AUDIO_END_ID - W0
N_CHUNK = 16
BAND_C2 = 45056
OFF_C2 = 20480

CH = 16384
NV = WW // 16
NVC = CH // 16
UNROLL = 8
NEG = float(np.finfo(np.float32).min)


def _lane_reduce(vec, op):
    acc = vec[0]
    for j in range(1, 16):
        acc = op(acc, vec[j])
    return acc


def _ukey(z):
    bits = lax.bitcast_convert_type(z, jnp.int32) | np.int32(-2147483648)
    return bits ^ np.int32(-1)


def _walk(hist, sums, rem, nvreg):
    f32 = jnp.float32

    def suma_body(i, _):
        sums[i] = plsc.cumsum(hist[pl.ds(i * 16, 16)])[15]
        return 0

    lax.fori_loop(0, nvreg, suma_body, 0)

    def walk_body(i, carry):
        sel, above, run = carry
        j = np.int32(nvreg - 1) - i
        run2 = run + sums[j]
        cross = (run2 > rem) & (sel < 0)
        sel = jnp.where(cross, j, sel)
        above = jnp.where(cross, run, above)
        return sel, above, run2

    sel, above, _ = lax.fori_loop(0, nvreg, walk_body,
                                  (jnp.int32(-1), f32(0.0), f32(0.0)))
    sel = jnp.maximum(sel, 0)
    rem2 = rem - above

    v = hist[pl.ds(sel * 16, 16)]
    rv = lax.rev(v, (0,))
    cs = plsc.cumsum(rv)
    cond = cs > rem2
    cnt = _lane_reduce(jnp.where(cond, jnp.int32(1), jnp.int32(0)),
                       lambda a, b: a + b)
    cnt = jnp.maximum(cnt, 1)
    binv = sel * 16 + (cnt - 1)
    above2 = _lane_reduce(jnp.where(cond, f32(0.0), rv), lambda a, b: a + b)
    return binv, rem2 - above2


def _zero_hist(hist, nvreg):
    def body(i, _):
        hist[pl.ds(i * 16, 16)] = jnp.zeros((16,), jnp.float32)
        return 0
    lax.fori_loop(0, nvreg, body, 0)


def _sc_body(cond_hbm, uncond_hbm, bprobs_hbm, cfg2_hbm, arr, cw, uw, ob, hist,
             sums):
    f32 = jnp.float32
    wid = lax.axis_index("s") * 2 + lax.axis_index("c")

    @pl.when(wid < B)
    def _row():
        row = wid
        iota = lax.iota(jnp.int32, 16)
        inv_t = f32(1.0 / TEMPERATURE)

        def pa_chunk(k, mx):
            pltpu.sync_copy(cond_hbm.at[row, pl.ds(k * CH, CH)], cw)
            pltpu.sync_copy(uncond_hbm.at[row, pl.ds(k * CH, CH)], uw)

            def body(off, mxi):
                cvec = cw[pl.ds(off, 16)]
                uvec = uw[pl.ds(off, 16)]
                cfg = uvec + f32(CFG_SCALE) * (cvec - uvec)
                ob[pl.ds(off, 16)] = cfg / f32(TEMPERATURE)
                col = iota + (k * CH + off)
                act = (col == EOS_I) | ((col >= A0_I) & (col < A1_I))
                cfgm = jnp.where(act, cfg, f32(NEG))
                arr[pl.ds(k * CH + off, 16)] = cfgm
                return jnp.maximum(mxi, cfgm)

            mx = plsc.parallel_loop(0, CH, 16, unroll=UNROLL, carry=mx)(body)
            pltpu.sync_copy(ob, cfg2_hbm.at[row, pl.ds(k * CH, CH)])
            return mx

        mxv = lax.fori_loop(0, 4, pa_chunk, jnp.full((16,), NEG, f32))
        m = _lane_reduce(mxv, jnp.maximum)

        _zero_hist(hist, 64)

        def pb_body(off, z):
            zvv = arr[pl.ds(off, 16)] - m
            arr[pl.ds(off, 16)] = zvv
            pv = jnp.exp(zvv)
            idx = lax.shift_right_logical(_ukey(zvv), np.int32(21))
            plsc.addupdate_scatter(hist, [idx], pv)
            return z + pv

        zv = plsc.parallel_loop(0, WW, 16, unroll=UNROLL,
                                carry=jnp.zeros((16,), f32))(pb_body)
        rem = f32(TOP_P) * _lane_reduce(zv, lambda a, b: a + b)
        b1, rem = _walk(hist, sums, rem, 64)

        _zero_hist(hist, 64)

        def r2_body(off):
            uk = _ukey(arr[pl.ds(off, 16)])
            pv = jnp.exp(arr[pl.ds(off, 16)])
            msk = lax.shift_right_logical(uk, np.int32(21)) == b1
            idx = lax.shift_right_logical(uk, np.int32(11)) & np.int32(0x3FF)
            plsc.addupdate_scatter(hist, [idx], pv, mask=msk)

        plsc.parallel_loop(0, WW, 16, unroll=UNROLL)(r2_body)
        b2, rem = _walk(hist, sums, rem, 64)

        pre = lax.shift_left(b1, np.int32(10)) | b2
        _zero_hist(hist, 128)

        def r3_body(off):
            uk = _ukey(arr[pl.ds(off, 16)])
            pv = jnp.exp(arr[pl.ds(off, 16)])
            msk = lax.shift_right_logical(uk, np.int32(11)) == pre
            idx = uk & np.int32(0x7FF)
            plsc.addupdate_scatter(hist, [idx], pv, mask=msk)

        plsc.parallel_loop(0, WW, 16, unroll=UNROLL)(r3_body)
        b3, _ = _walk(hist, sums, rem, 128)
        thr = (lax.shift_left(b1, np.int32(21))
               | lax.shift_left(b2, np.int32(11)) | b3)

        def pc_body(off, z2):
            zvv = arr[pl.ds(off, 16)]
            keep = _ukey(zvv) >= thr
            p2 = jnp.where(keep, jnp.exp(zvv * inv_t), f32(0.0))
            arr[pl.ds(off, 16)] = p2
            return z2 + p2

        z2v = plsc.parallel_loop(0, WW, 16, unroll=UNROLL,
                                 carry=jnp.zeros((16,), f32))(pc_body)
        z2 = _lane_reduce(z2v, lambda a, b: a + b)

        def pd_body(off):
            arr[pl.ds(off, 16)] = arr[pl.ds(off, 16)] / z2

        plsc.parallel_loop(0, WW, 16, unroll=UNROLL)(pd_body)

        def pd_out(k, _):
            pltpu.sync_copy(arr.at[pl.ds(k * CH, CH)],
                            bprobs_hbm.at[row, pl.ds(k * CH, CH)])
            return 0

        lax.fori_loop(0, 4, pd_out, 0)


def _sc_topp(cond_w, uncond_w):
    mesh = plsc.VectorSubcoreMesh(core_axis_name="c", subcore_axis_name="s",
                                  num_cores=2, num_subcores=16)
    fn = pl.kernel(
        _sc_body,
        compiler_params=pltpu.CompilerParams(needs_layout_passes=False),
        out_type=[
            jax.ShapeDtypeStruct((B, WW), jnp.float32),
            jax.ShapeDtypeStruct((B, WW), jnp.float32),
        ],
        mesh=mesh,
        scratch_types=[
            pltpu.VMEM((WW,), jnp.float32),
            pltpu.VMEM((CH,), jnp.float32),
            pltpu.VMEM((CH,), jnp.float32),
            pltpu.VMEM((CH,), jnp.float32),
            pltpu.VMEM((2048,), jnp.float32),
            pltpu.SMEM((128,), jnp.float32),
        ],
    )
    return fn(cond_w, uncond_w)


def _rotl(x, d):
    return lax.shift_left(x, np.int32(d)) | lax.shift_right_logical(x, np.int32(32 - d))


def _threefry(x0, x1):
    ks0 = np.int32(0)
    ks1 = np.int32(SAMPLE_SEED)
    ks2 = np.int32(ks0 ^ ks1 ^ np.int32(0x1BD11BDA))
    rot1 = (13, 15, 26, 6)
    rot2 = (17, 29, 16, 24)
    x0 = x0 + ks0
    x1 = x1 + ks1
    for r in rot1:
        x0 = x0 + x1; x1 = _rotl(x1, r); x1 = x0 ^ x1
    x0 = x0 + ks1; x1 = x1 + ks2 + np.int32(1)
    for r in rot2:
        x0 = x0 + x1; x1 = _rotl(x1, r); x1 = x0 ^ x1
    x0 = x0 + ks2; x1 = x1 + ks0 + np.int32(2)
    for r in rot1:
        x0 = x0 + x1; x1 = _rotl(x1, r); x1 = x0 ^ x1
    x0 = x0 + ks0; x1 = x1 + ks1 + np.int32(3)
    for r in rot2:
        x0 = x0 + x1; x1 = _rotl(x1, r); x1 = x0 ^ x1
    x0 = x0 + ks1; x1 = x1 + ks2 + np.int32(4)
    for r in rot1:
        x0 = x0 + x1; x1 = _rotl(x1, r); x1 = x0 ^ x1
    x0 = x0 + ks2; x1 = x1 + ks0 + np.int32(5)
    return x0, x1


def _gumbel_window():
    row = lax.broadcasted_iota(jnp.int32, (B, WW), 0)
    col = lax.broadcasted_iota(jnp.int32, (B, WW), 1)
    flat = row * np.int32(V) + (col + np.int32(W0))
    o1, o2 = _threefry(jnp.zeros((B, WW), jnp.int32), flat)
    bits = o1 ^ o2
    fb = lax.shift_right_logical(bits, np.int32(9)) | np.int32(0x3F800000)
    f = lax.bitcast_convert_type(fb, jnp.float32) - jnp.float32(1.0)
    tiny = jnp.float32(np.finfo(np.float32).tiny)
    u = jnp.maximum(tiny, f * (jnp.float32(1.0) - tiny) + tiny)
    return -jnp.log(-jnp.log(u))


def _tc_kernel(bprobs_ref, cfg2_ref, probs_ref, ntok_ref):
    c = pl.program_id(0)

    @pl.when(c == 0)
    def _sample():
        col = lax.broadcasted_iota(jnp.int32, (B, WW), 1)
        g = _gumbel_window()
        keep = bprobs_ref[...] > 0.0
        score = jnp.where(keep, cfg2_ref[...] + g, -jnp.inf)
        smax = jnp.max(score, axis=1, keepdims=True)
        win = jnp.where(score == smax, col, np.int32(2 * WW))
        idx = jnp.min(win, axis=1, keepdims=True) + np.int32(W0)
        ntok_ref[...] = jnp.broadcast_to(idx, (B, 128))

    probs_ref[...] = jnp.zeros((B, WW), jnp.float32)

    @pl.when(c == 2)
    def _band_lo():
        probs_ref[:, OFF_C2:] = bprobs_ref[:, :BAND_C2]

    @pl.when(c == 3)
    def _band_hi():
        probs_ref[:, :OFF_C2] = bprobs_ref[:, BAND_C2:]


def kernel(cond_logits, uncond_logits):
    cond_w = lax.slice(cond_logits, (0, W0), (B, W0 + WW))
    uncond_w = lax.slice(uncond_logits, (0, W0), (B, W0 + WW))
    bprobs, cfg2 = _sc_topp(cond_w, uncond_w)
    probs, ntok = pl.pallas_call(
        _tc_kernel,
        grid=(N_CHUNK,),
        in_specs=[
            pl.BlockSpec((B, WW), lambda c: (0, 0)),
            pl.BlockSpec((B, WW), lambda c: (0, 0)),
        ],
        out_specs=[
            pl.BlockSpec((B, WW), lambda c: (0, c)),
            pl.BlockSpec((B, 128), lambda c: (0, 0)),
        ],
        out_shape=[
            jax.ShapeDtypeStruct((B, V), jnp.float32),
            jax.ShapeDtypeStruct((B, 128), jnp.int32),
        ],
    )(bprobs, cfg2)
    return probs, ntok[:, 0]


# speedup vs baseline: 2.8383x; 1.0507x over previous
import functools
import jax
import jax.numpy as jnp
import numpy as np
from jax import lax
from jax.experimental import pallas as pl
from jax.experimental.pallas import tpu as pltpu
from jax.experimental.pallas import tpu_sc as plsc

CFG_SCALE = 2.0
TEMPERATURE = 0.85
TOP_P = 0.9
AUDIO_START_ID = 151669
AUDIO_END_ID = 215669
EOS_TOKEN_ID = 151645
SAMPLE_SEED = 1
B = 8
V = 1000000

W0 = 151552
WW = 65536
EOS_I = EOS_TOKEN_ID - W0
A0_I = AUDIO_START_ID - W0
A1_I = AUDIO_END_ID - W0
N_CHUNK = 16
BAND_C2 = 45056
OFF_C2 = 20480

CH = 16384
NV = WW // 16
NVC = CH // 16
UNROLL = 8
NEG = float(np.finfo(np.float32).min)


def _lane_reduce(vec, op):
    acc = vec[0]
    for j in range(1, 16):
        acc = op(acc, vec[j])
    return acc


def _ukey(z):
    bits = lax.bitcast_convert_type(z, jnp.int32) | np.int32(-2147483648)
    return bits ^ np.int32(-1)


def _walk(hist, sums, rem, nvreg):
    f32 = jnp.float32

    def suma_body(i, _):
        sums[i] = plsc.cumsum(hist[pl.ds(i * 16, 16)])[15]
        return 0

    lax.fori_loop(0, nvreg, suma_body, 0)

    def walk_body(i, carry):
        sel, above, run = carry
        j = np.int32(nvreg - 1) - i
        run2 = run + sums[j]
        cross = (run2 > rem) & (sel < 0)
        sel = jnp.where(cross, j, sel)
        above = jnp.where(cross, run, above)
        return sel, above, run2

    sel, above, _ = lax.fori_loop(0, nvreg, walk_body,
                                  (jnp.int32(-1), f32(0.0), f32(0.0)))
    sel = jnp.maximum(sel, 0)
    rem2 = rem - above

    v = hist[pl.ds(sel * 16, 16)]
    rv = lax.rev(v, (0,))
    cs = plsc.cumsum(rv)
    cond = cs > rem2
    cnt = _lane_reduce(jnp.where(cond, jnp.int32(1), jnp.int32(0)),
                       lambda a, b: a + b)
    cnt = jnp.maximum(cnt, 1)
    binv = sel * 16 + (cnt - 1)
    above2 = _lane_reduce(jnp.where(cond, f32(0.0), rv), lambda a, b: a + b)
    return binv, rem2 - above2


def _zero_hist(hist, nvreg):
    def body(i, _):
        hist[pl.ds(i * 16, 16)] = jnp.zeros((16,), jnp.float32)
        return 0
    lax.fori_loop(0, nvreg, body, 0)


def _sc_body(cond_hbm, uncond_hbm, bprobs_hbm, cfg2_hbm, arr, cw, uw, ob, hist,
             sums):
    f32 = jnp.float32
    wid = lax.axis_index("s") * 2 + lax.axis_index("c")

    @pl.when(wid < B)
    def _row():
        row = wid
        iota = lax.iota(jnp.int32, 16)
        inv_t = f32(1.0 / TEMPERATURE)

        def pa_chunk(k, mx):
            pltpu.sync_copy(cond_hbm.at[row, pl.ds(k * CH, CH)], cw)
            pltpu.sync_copy(uncond_hbm.at[row, pl.ds(k * CH, CH)], uw)

            def body(off, mxi):
                cvec = cw[pl.ds(off, 16)]
                uvec = uw[pl.ds(off, 16)]
                cfg = uvec + f32(CFG_SCALE) * (cvec - uvec)
                ob[pl.ds(off, 16)] = cfg / f32(TEMPERATURE)
                col = iota + (k * CH + off)
                act = (col == EOS_I) | ((col >= A0_I) & (col < A1_I))
                cfgm = jnp.where(act, cfg, f32(NEG))
                arr[pl.ds(k * CH + off, 16)] = cfgm
                return jnp.maximum(mxi, cfgm)

            mx = plsc.parallel_loop(0, CH, 16, unroll=UNROLL, carry=mx)(body)
            pltpu.sync_copy(ob, cfg2_hbm.at[row, pl.ds(k * CH, CH)])
            return mx

        mxv = lax.fori_loop(0, 4, pa_chunk, jnp.full((16,), NEG, f32))
        m = _lane_reduce(mxv, jnp.maximum)

        _zero_hist(hist, 64)

        def pb_body(off, z):
            zvv = arr[pl.ds(off, 16)] - m
            arr[pl.ds(off, 16)] = zvv
            pv = jnp.exp(zvv)
            idx = lax.shift_right_logical(_ukey(zvv), np.int32(21))
            plsc.addupdate_scatter(hist, [idx], pv)
            return z + pv

        zv = plsc.parallel_loop(0, WW, 16, unroll=UNROLL,
                                carry=jnp.zeros((16,), f32))(pb_body)
        rem = f32(TOP_P) * _lane_reduce(zv, lambda a, b: a + b)
        b1, rem = _walk(hist, sums, rem, 64)

        _zero_hist(hist, 64)

        def r2_body(off):
            uk = _ukey(arr[pl.ds(off, 16)])
            pv = jnp.exp(arr[pl.ds(off, 16)])
            msk = lax.shift_right_logical(uk, np.int32(21)) == b1
            idx = lax.shift_right_logical(uk, np.int32(11)) & np.int32(0x3FF)
            plsc.addupdate_scatter(hist, [idx], pv, mask=msk)

        plsc.parallel_loop(0, WW, 16, unroll=UNROLL)(r2_body)
        b2, rem = _walk(hist, sums, rem, 64)

        pre = lax.shift_left(b1, np.int32(10)) | b2
        _zero_hist(hist, 128)

        def r3_body(off):
            uk = _ukey(arr[pl.ds(off, 16)])
            pv = jnp.exp(arr[pl.ds(off, 16)])
            msk = lax.shift_right_logical(uk, np.int32(11)) == pre
            idx = uk & np.int32(0x7FF)
            plsc.addupdate_scatter(hist, [idx], pv, mask=msk)

        plsc.parallel_loop(0, WW, 16, unroll=UNROLL)(r3_body)
        b3, _ = _walk(hist, sums, rem, 128)
        thr = (lax.shift_left(b1, np.int32(21))
               | lax.shift_left(b2, np.int32(11)) | b3)

        def pc_body(off, z2):
            zvv = arr[pl.ds(off, 16)]
            keep = _ukey(zvv) >= thr
            p2 = jnp.where(keep, jnp.exp(zvv * inv_t), f32(0.0))
            arr[pl.ds(off, 16)] = p2
            return z2 + p2

        z2v = plsc.parallel_loop(0, WW, 16, unroll=UNROLL,
                                 carry=jnp.zeros((16,), f32))(pc_body)
        z2 = _lane_reduce(z2v, lambda a, b: a + b)

        def pd_body(off):
            arr[pl.ds(off, 16)] = arr[pl.ds(off, 16)] / z2

        plsc.parallel_loop(0, WW, 16, unroll=UNROLL)(pd_body)

        def pd_out(k, _):
            pltpu.sync_copy(arr.at[pl.ds(k * CH, CH)],
                            bprobs_hbm.at[row, pl.ds(k * CH, CH)])
            return 0

        lax.fori_loop(0, 4, pd_out, 0)


def _sc_topp(cond_w, uncond_w):
    mesh = plsc.VectorSubcoreMesh(core_axis_name="c", subcore_axis_name="s",
                                  num_cores=2, num_subcores=16)
    fn = pl.kernel(
        _sc_body,
        compiler_params=pltpu.CompilerParams(needs_layout_passes=False),
        out_type=[
            jax.ShapeDtypeStruct((B, WW), jnp.float32),
            jax.ShapeDtypeStruct((B, WW), jnp.float32),
        ],
        mesh=mesh,
        scratch_types=[
            pltpu.VMEM((WW,), jnp.float32),
            pltpu.VMEM((CH,), jnp.float32),
            pltpu.VMEM((CH,), jnp.float32),
            pltpu.VMEM((CH,), jnp.float32),
            pltpu.VMEM((2048,), jnp.float32),
            pltpu.SMEM((128,), jnp.float32),
        ],
    )
    return fn(cond_w, uncond_w)


def _rotl(x, d):
    return lax.shift_left(x, np.int32(d)) | lax.shift_right_logical(x, np.int32(32 - d))


def _threefry(x0, x1):
    ks0 = np.int32(0)
    ks1 = np.int32(SAMPLE_SEED)
    ks2 = np.int32(ks0 ^ ks1 ^ np.int32(0x1BD11BDA))
    rot1 = (13, 15, 26, 6)
    rot2 = (17, 29, 16, 24)
    x0 = x0 + ks0
    x1 = x1 + ks1
    for r in rot1:
        x0 = x0 + x1; x1 = _rotl(x1, r); x1 = x0 ^ x1
    x0 = x0 + ks1; x1 = x1 + ks2 + np.int32(1)
    for r in rot2:
        x0 = x0 + x1; x1 = _rotl(x1, r); x1 = x0 ^ x1
    x0 = x0 + ks2; x1 = x1 + ks0 + np.int32(2)
    for r in rot1:
        x0 = x0 + x1; x1 = _rotl(x1, r); x1 = x0 ^ x1
    x0 = x0 + ks0; x1 = x1 + ks1 + np.int32(3)
    for r in rot2:
        x0 = x0 + x1; x1 = _rotl(x1, r); x1 = x0 ^ x1
    x0 = x0 + ks1; x1 = x1 + ks2 + np.int32(4)
    for r in rot1:
        x0 = x0 + x1; x1 = _rotl(x1, r); x1 = x0 ^ x1
    x0 = x0 + ks2; x1 = x1 + ks0 + np.int32(5)
    return x0, x1


def _gumbel_window():
    row = lax.broadcasted_iota(jnp.int32, (B, WW), 0)
    col = lax.broadcasted_iota(jnp.int32, (B, WW), 1)
    flat = row * np.int32(V) + (col + np.int32(W0))
    o1, o2 = _threefry(jnp.zeros((B, WW), jnp.int32), flat)
    bits = o1 ^ o2
    fb = lax.shift_right_logical(bits, np.int32(9)) | np.int32(0x3F800000)
    f = lax.bitcast_convert_type(fb, jnp.float32) - jnp.float32(1.0)
    tiny = jnp.float32(np.finfo(np.float32).tiny)
    u = jnp.maximum(tiny, f * (jnp.float32(1.0) - tiny) + tiny)
    return -jnp.log(-jnp.log(u))


def _zero_kernel(probs_ref):
    probs_ref[...] = jnp.zeros((B, WW), jnp.float32)


def _tc_kernel(zbuf_ref, bprobs_ref, cfg2_ref, probs_ref, ntok_ref):
    del zbuf_ref
    col = lax.broadcasted_iota(jnp.int32, (B, WW), 1)
    g = _gumbel_window()
    keep = bprobs_ref[...] > 0.0
    score = jnp.where(keep, cfg2_ref[...] + g, -jnp.inf)
    smax = jnp.max(score, axis=1, keepdims=True)
    win = jnp.where(score == smax, col, np.int32(2 * WW))
    idx = jnp.min(win, axis=1, keepdims=True) + np.int32(W0)
    ntok_ref[...] = jnp.broadcast_to(idx, (B, 128))
    pltpu.sync_copy(bprobs_ref, probs_ref.at[:, pl.ds(W0, WW)])


def kernel(cond_logits, uncond_logits):
    cond_w = lax.slice(cond_logits, (0, W0), (B, W0 + WW))
    uncond_w = lax.slice(uncond_logits, (0, W0), (B, W0 + WW))
    zbuf = pl.pallas_call(
        _zero_kernel,
        grid=(N_CHUNK,),
        out_specs=pl.BlockSpec((B, WW), lambda c: (0, c)),
        out_shape=jax.ShapeDtypeStruct((B, V), jnp.float32),
    )()
    bprobs, cfg2 = _sc_topp(cond_w, uncond_w)
    probs, ntok = pl.pallas_call(
        _tc_kernel,
        in_specs=[
            pl.BlockSpec(memory_space=pl.ANY),
            pl.BlockSpec((B, WW), lambda: (0, 0)),
            pl.BlockSpec((B, WW), lambda: (0, 0)),
        ],
        out_specs=[
            pl.BlockSpec(memory_space=pl.ANY),
            pl.BlockSpec((B, 128), lambda: (0, 0)),
        ],
        out_shape=[
            jax.ShapeDtypeStruct((B, V), jnp.float32),
            jax.ShapeDtypeStruct((B, 128), jnp.int32),
        ],
        input_output_aliases={0: 0},
    )(zbuf, bprobs, cfg2)
    return probs, ntok[:, 0]


# speedup vs baseline: 2.9295x; 1.0321x over previous
import functools
import jax
import jax.numpy as jnp
import numpy as np
from jax import lax
from jax.experimental import pallas as pl
from jax.experimental.pallas import tpu as pltpu
from jax.experimental.pallas import tpu_sc as plsc

CFG_SCALE = 2.0
TEMPERATURE = 0.85
TOP_P = 0.9
AUDIO_START_ID = 151669
AUDIO_END_ID = 215669
EOS_TOKEN_ID = 151645
SAMPLE_SEED = 1
B = 8
V = 1000000

W0 = 151552
WW = 65536
EOS_I = EOS_TOKEN_ID - W0
A0_I = AUDIO_START_ID - W0
A1_I = AUDIO_END_ID - W0
N_CHUNK = 16
BAND_C2 = 45056
OFF_C2 = 20480

CH = 16384
NV = WW // 16
NVC = CH // 16
UNROLL = 8
NEG = float(np.finfo(np.float32).min)


def _lane_reduce(vec, op):
    acc = vec[0]
    for j in range(1, 16):
        acc = op(acc, vec[j])
    return acc


def _ukey(z):
    bits = lax.bitcast_convert_type(z, jnp.int32) | np.int32(-2147483648)
    return bits ^ np.int32(-1)


def _walk(hist, sums, rem, nvreg):
    f32 = jnp.float32

    def suma_body(i, _):
        sums[i] = plsc.cumsum(hist[pl.ds(i * 16, 16)])[15]
        return 0

    lax.fori_loop(0, nvreg, suma_body, 0)

    if rem is None:
        def tot_body(i, t):
            return t + sums[i]
        rem = f32(TOP_P) * lax.fori_loop(0, nvreg, tot_body, f32(0.0))

    def walk_body(i, carry):
        sel, above, run = carry
        j = np.int32(nvreg - 1) - i
        run2 = run + sums[j]
        cross = (run2 > rem) & (sel < 0)
        sel = jnp.where(cross, j, sel)
        above = jnp.where(cross, run, above)
        return sel, above, run2

    sel, above, _ = lax.fori_loop(0, nvreg, walk_body,
                                  (jnp.int32(-1), f32(0.0), f32(0.0)))
    sel = jnp.maximum(sel, 0)
    rem2 = rem - above

    v = hist[pl.ds(sel * 16, 16)]
    rv = lax.rev(v, (0,))
    cs = plsc.cumsum(rv)
    cond = cs > rem2
    cnt = _lane_reduce(jnp.where(cond, jnp.int32(1), jnp.int32(0)),
                       lambda a, b: a + b)
    cnt = jnp.maximum(cnt, 1)
    binv = sel * 16 + (cnt - 1)
    above2 = _lane_reduce(jnp.where(cond, f32(0.0), rv), lambda a, b: a + b)
    return binv, rem2 - above2


def _zero_hist(hist, nvreg):
    def body(i, _):
        hist[pl.ds(i * 16, 16)] = jnp.zeros((16,), jnp.float32)
        return 0
    lax.fori_loop(0, nvreg, body, 0)


def _sc_body(cond_hbm, uncond_hbm, bprobs_hbm, arr, cw, uw, hist, sums):
    f32 = jnp.float32
    wid = lax.axis_index("s") * 2 + lax.axis_index("c")

    @pl.when(wid < B)
    def _row():
        row = wid
        iota = lax.iota(jnp.int32, 16)
        inv_t = f32(1.0 / TEMPERATURE)

        def pa_chunk(k, mx):
            pltpu.sync_copy(cond_hbm.at[row, pl.ds(k * CH, CH)], cw)
            pltpu.sync_copy(uncond_hbm.at[row, pl.ds(k * CH, CH)], uw)

            def body(off, mxi):
                cvec = cw[pl.ds(off, 16)]
                uvec = uw[pl.ds(off, 16)]
                cfg = uvec + f32(CFG_SCALE) * (cvec - uvec)
                col = iota + (k * CH + off)
                act = (col == EOS_I) | ((col >= A0_I) & (col < A1_I))
                cfgm = jnp.where(act, cfg, f32(NEG))
                arr[pl.ds(k * CH + off, 16)] = cfgm
                return jnp.maximum(mxi, cfgm)

            return plsc.parallel_loop(0, CH, 16, unroll=UNROLL, carry=mx)(body)

        mxv = lax.fori_loop(0, 4, pa_chunk, jnp.full((16,), NEG, f32))
        m = _lane_reduce(mxv, jnp.maximum)

        _zero_hist(hist, 64)

        def pb_body(off):
            zvv = arr[pl.ds(off, 16)] - m
            arr[pl.ds(off, 16)] = zvv
            pv = jnp.exp(zvv)
            idx = lax.shift_right_logical(_ukey(zvv), np.int32(21))
            plsc.addupdate_scatter(hist, [idx], pv)

        plsc.parallel_loop(0, WW, 16, unroll=UNROLL)(pb_body)
        b1, rem = _walk(hist, sums, None, 64)

        _zero_hist(hist, 64)

        def r2_body(off):
            uk = _ukey(arr[pl.ds(off, 16)])
            pv = jnp.exp(arr[pl.ds(off, 16)])
            msk = lax.shift_right_logical(uk, np.int32(21)) == b1
            idx = lax.shift_right_logical(uk, np.int32(11)) & np.int32(0x3FF)
            plsc.addupdate_scatter(hist, [idx], pv, mask=msk)

        plsc.parallel_loop(0, WW, 16, unroll=UNROLL)(r2_body)
        b2, rem = _walk(hist, sums, rem, 64)

        pre = lax.shift_left(b1, np.int32(10)) | b2
        _zero_hist(hist, 128)

        def r3_body(off):
            uk = _ukey(arr[pl.ds(off, 16)])
            pv = jnp.exp(arr[pl.ds(off, 16)])
            msk = lax.shift_right_logical(uk, np.int32(11)) == pre
            idx = uk & np.int32(0x7FF)
            plsc.addupdate_scatter(hist, [idx], pv, mask=msk)

        plsc.parallel_loop(0, WW, 16, unroll=UNROLL)(r3_body)
        b3, _ = _walk(hist, sums, rem, 128)
        thr = (lax.shift_left(b1, np.int32(21))
               | lax.shift_left(b2, np.int32(11)) | b3)

        def pc_body(off):
            zvv = arr[pl.ds(off, 16)]
            keep = _ukey(zvv) >= thr
            p2 = jnp.where(keep, jnp.exp(zvv * inv_t), f32(0.0))
            arr[pl.ds(off, 16)] = p2

        plsc.parallel_loop(0, WW, 16, unroll=UNROLL)(pc_body)

        def pd_out(k, _):
            pltpu.sync_copy(arr.at[pl.ds(k * CH, CH)],
                            bprobs_hbm.at[row, pl.ds(k * CH, CH)])
            return 0

        lax.fori_loop(0, 4, pd_out, 0)


def _sc_topp(cond_w, uncond_w):
    mesh = plsc.VectorSubcoreMesh(core_axis_name="c", subcore_axis_name="s",
                                  num_cores=2, num_subcores=16)
    fn = pl.kernel(
        _sc_body,
        compiler_params=pltpu.CompilerParams(needs_layout_passes=False),
        out_type=jax.ShapeDtypeStruct((B, WW), jnp.float32),
        mesh=mesh,
        scratch_types=[
            pltpu.VMEM((WW,), jnp.float32),
            pltpu.VMEM((CH,), jnp.float32),
            pltpu.VMEM((CH,), jnp.float32),
            pltpu.VMEM((2048,), jnp.float32),
            pltpu.SMEM((128,), jnp.float32),
        ],
    )
    return fn(cond_w, uncond_w)


def _rotl(x, d):
    return lax.shift_left(x, np.int32(d)) | lax.shift_right_logical(x, np.int32(32 - d))


def _threefry(x0, x1):
    ks0 = np.int32(0)
    ks1 = np.int32(SAMPLE_SEED)
    ks2 = np.int32(ks0 ^ ks1 ^ np.int32(0x1BD11BDA))
    rot1 = (13, 15, 26, 6)
    rot2 = (17, 29, 16, 24)
    x0 = x0 + ks0
    x1 = x1 + ks1
    for r in rot1:
        x0 = x0 + x1; x1 = _rotl(x1, r); x1 = x0 ^ x1
    x0 = x0 + ks1; x1 = x1 + ks2 + np.int32(1)
    for r in rot2:
        x0 = x0 + x1; x1 = _rotl(x1, r); x1 = x0 ^ x1
    x0 = x0 + ks2; x1 = x1 + ks0 + np.int32(2)
    for r in rot1:
        x0 = x0 + x1; x1 = _rotl(x1, r); x1 = x0 ^ x1
    x0 = x0 + ks0; x1 = x1 + ks1 + np.int32(3)
    for r in rot2:
        x0 = x0 + x1; x1 = _rotl(x1, r); x1 = x0 ^ x1
    x0 = x0 + ks1; x1 = x1 + ks2 + np.int32(4)
    for r in rot1:
        x0 = x0 + x1; x1 = _rotl(x1, r); x1 = x0 ^ x1
    x0 = x0 + ks2; x1 = x1 + ks0 + np.int32(5)
    return x0, x1


def _gumbel_window():
    row = lax.broadcasted_iota(jnp.int32, (B, WW), 0)
    col = lax.broadcasted_iota(jnp.int32, (B, WW), 1)
    flat = row * np.int32(V) + (col + np.int32(W0))
    o1, o2 = _threefry(jnp.zeros((B, WW), jnp.int32), flat)
    bits = o1 ^ o2
    fb = lax.shift_right_logical(bits, np.int32(9)) | np.int32(0x3F800000)
    f = lax.bitcast_convert_type(fb, jnp.float32) - jnp.float32(1.0)
    tiny = jnp.float32(np.finfo(np.float32).tiny)
    u = jnp.maximum(tiny, f * (jnp.float32(1.0) - tiny) + tiny)
    return -jnp.log(-jnp.log(u))


def _zero_kernel(probs_ref):
    probs_ref[...] = jnp.zeros((B, WW), jnp.float32)


def _tc_kernel(zbuf_ref, cond_ref, uncond_ref, bprobs_ref, probs_ref, ntok_ref,
               band_vmem):
    del zbuf_ref
    col = lax.broadcasted_iota(jnp.int32, (B, WW), 1)
    p2 = bprobs_ref[...]
    keep = p2 > 0.0
    z2 = jnp.sum(p2, axis=1, keepdims=True)
    band_vmem[...] = p2 / z2
    pltpu.sync_copy(band_vmem, probs_ref.at[:, pl.ds(W0, WW)])

    cw = cond_ref[...]
    uw = uncond_ref[...]
    cfg2 = (uw + jnp.float32(CFG_SCALE) * (cw - uw)) / jnp.float32(TEMPERATURE)
    g = _gumbel_window()
    score = jnp.where(keep, cfg2 + g, -jnp.inf)
    smax = jnp.max(score, axis=1, keepdims=True)
    win = jnp.where(score == smax, col, np.int32(2 * WW))
    idx = jnp.min(win, axis=1, keepdims=True) + np.int32(W0)
    ntok_ref[...] = jnp.broadcast_to(idx, (B, 128))


def kernel(cond_logits, uncond_logits):
    cond_w = lax.slice(cond_logits, (0, W0), (B, W0 + WW))
    uncond_w = lax.slice(uncond_logits, (0, W0), (B, W0 + WW))
    zbuf = pl.pallas_call(
        _zero_kernel,
        grid=(N_CHUNK,),
        out_specs=pl.BlockSpec((B, WW), lambda c: (0, c)),
        out_shape=jax.ShapeDtypeStruct((B, V), jnp.float32),
    )()
    bprobs = _sc_topp(cond_w, uncond_w)
    probs, ntok = pl.pallas_call(
        _tc_kernel,
        in_specs=[
            pl.BlockSpec(memory_space=pl.ANY),
            pl.BlockSpec((B, WW), lambda: (0, 0)),
            pl.BlockSpec((B, WW), lambda: (0, 0)),
            pl.BlockSpec((B, WW), lambda: (0, 0)),
        ],
        out_specs=[
            pl.BlockSpec(memory_space=pl.ANY),
            pl.BlockSpec((B, 128), lambda: (0, 0)),
        ],
        out_shape=[
            jax.ShapeDtypeStruct((B, V), jnp.float32),
            jax.ShapeDtypeStruct((B, 128), jnp.int32),
        ],
        scratch_shapes=[pltpu.VMEM((B, WW), jnp.float32)],
        input_output_aliases={0: 0},
    )(zbuf, cond_w, uncond_w, bprobs)
    return probs, ntok[:, 0]


# speedup vs baseline: 3.0416x; 1.0383x over previous
import functools
import jax
import jax.numpy as jnp
import numpy as np
from jax import lax
from jax.experimental import pallas as pl
from jax.experimental.pallas import tpu as pltpu
from jax.experimental.pallas import tpu_sc as plsc

CFG_SCALE = 2.0
TEMPERATURE = 0.85
TOP_P = 0.9
AUDIO_START_ID = 151669
AUDIO_END_ID = 215669
EOS_TOKEN_ID = 151645
SAMPLE_SEED = 1
B = 8
V = 1000000

W0 = 151552
WW = 65536
EOS_I = EOS_TOKEN_ID - W0
A0_I = AUDIO_START_ID - W0
A1_I = AUDIO_END_ID - W0
N_CHUNK = 16
BAND_C2 = 45056
OFF_C2 = 20480

CH = 16384
HH = WW // 2
NV = WW // 16
NVC = CH // 16
UNROLL = 8
NEG = float(np.finfo(np.float32).min)


def _lane_reduce(vec, op):
    acc = vec[0]
    for j in range(1, 16):
        acc = op(acc, vec[j])
    return acc


def _ukey(z):
    bits = lax.bitcast_convert_type(z, jnp.int32) | np.int32(-2147483648)
    return bits ^ np.int32(-1)


def _walk(hist, sums, rem, nvreg):
    f32 = jnp.float32

    def suma_body(i, _):
        sums[i] = plsc.cumsum(hist[pl.ds(i * 16, 16)])[15]
        return 0

    lax.fori_loop(0, nvreg, suma_body, 0)

    if rem is None:
        def tot_body(i, t):
            return t + sums[i]
        rem = f32(TOP_P) * lax.fori_loop(0, nvreg, tot_body, f32(0.0))

    def walk_body(i, carry):
        sel, above, run = carry
        j = np.int32(nvreg - 1) - i
        run2 = run + sums[j]
        cross = (run2 > rem) & (sel < 0)
        sel = jnp.where(cross, j, sel)
        above = jnp.where(cross, run, above)
        return sel, above, run2

    sel, above, _ = lax.fori_loop(0, nvreg, walk_body,
                                  (jnp.int32(-1), f32(0.0), f32(0.0)))
    sel = jnp.maximum(sel, 0)
    rem2 = rem - above

    v = hist[pl.ds(sel * 16, 16)]
    rv = lax.rev(v, (0,))
    cs = plsc.cumsum(rv)
    cond = cs > rem2
    cnt = _lane_reduce(jnp.where(cond, jnp.int32(1), jnp.int32(0)),
                       lambda a, b: a + b)
    cnt = jnp.maximum(cnt, 1)
    binv = sel * 16 + (cnt - 1)
    above2 = _lane_reduce(jnp.where(cond, f32(0.0), rv), lambda a, b: a + b)
    return binv, rem2 - above2


def _zero_hist(hist, nvreg):
    def body(i, _):
        hist[pl.ds(i * 16, 16)] = jnp.zeros((16,), jnp.float32)
        return 0
    lax.fori_loop(0, nvreg, body, 0)


def _sc_body(cond_hbm, uncond_hbm, bprobs_hbm, arr, ub, hist, sums):
    f32 = jnp.float32
    wid = lax.axis_index("s") * 2 + lax.axis_index("c")

    @pl.when(wid < B)
    def _row():
        row = wid
        iota = lax.iota(jnp.int32, 16)
        inv_t = f32(1.0 / TEMPERATURE)

        pltpu.sync_copy(cond_hbm.at[row], arr)

        def pa_half(h, mx):
            pltpu.sync_copy(uncond_hbm.at[row, pl.ds(h * HH, HH)], ub)

            def body(off, mxi):
                cvec = arr[pl.ds(h * HH + off, 16)]
                uvec = ub[pl.ds(off, 16)]
                cfg = uvec + f32(CFG_SCALE) * (cvec - uvec)
                col = iota + (h * HH + off)
                act = (col == EOS_I) | ((col >= A0_I) & (col < A1_I))
                cfgm = jnp.where(act, cfg, f32(NEG))
                arr[pl.ds(h * HH + off, 16)] = cfgm
                return jnp.maximum(mxi, cfgm)

            return plsc.parallel_loop(0, HH, 16, unroll=UNROLL, carry=mx)(body)

        mxv = lax.fori_loop(0, 2, pa_half, jnp.full((16,), NEG, f32))
        m = _lane_reduce(mxv, jnp.maximum)

        _zero_hist(hist, 64)

        def pb_body(off):
            zvv = arr[pl.ds(off, 16)] - m
            arr[pl.ds(off, 16)] = zvv
            pv = jnp.exp(zvv)
            idx = lax.shift_right_logical(_ukey(zvv), np.int32(21))
            plsc.addupdate_scatter(hist, [idx], pv)

        plsc.parallel_loop(0, WW, 16, unroll=UNROLL)(pb_body)
        b1, rem = _walk(hist, sums, None, 64)

        _zero_hist(hist, 64)

        def r2_body(off):
            uk = _ukey(arr[pl.ds(off, 16)])
            pv = jnp.exp(arr[pl.ds(off, 16)])
            msk = lax.shift_right_logical(uk, np.int32(21)) == b1
            idx = lax.shift_right_logical(uk, np.int32(11)) & np.int32(0x3FF)
            plsc.addupdate_scatter(hist, [idx], pv, mask=msk)

        plsc.parallel_loop(0, WW, 16, unroll=UNROLL)(r2_body)
        b2, rem = _walk(hist, sums, rem, 64)

        pre = lax.shift_left(b1, np.int32(10)) | b2
        _zero_hist(hist, 128)

        def r3_body(off):
            uk = _ukey(arr[pl.ds(off, 16)])
            pv = jnp.exp(arr[pl.ds(off, 16)])
            msk = lax.shift_right_logical(uk, np.int32(11)) == pre
            idx = uk & np.int32(0x7FF)
            plsc.addupdate_scatter(hist, [idx], pv, mask=msk)

        plsc.parallel_loop(0, WW, 16, unroll=UNROLL)(r3_body)
        b3, _ = _walk(hist, sums, rem, 128)
        thr = (lax.shift_left(b1, np.int32(21))
               | lax.shift_left(b2, np.int32(11)) | b3)

        def pc_body(off):
            zvv = arr[pl.ds(off, 16)]
            keep = _ukey(zvv) >= thr
            p2 = jnp.where(keep, jnp.exp(zvv * inv_t), f32(0.0))
            arr[pl.ds(off, 16)] = p2

        plsc.parallel_loop(0, WW, 16, unroll=UNROLL)(pc_body)
        pltpu.sync_copy(arr, bprobs_hbm.at[row])


def _sc_topp(cond_w, uncond_w):
    mesh = plsc.VectorSubcoreMesh(core_axis_name="c", subcore_axis_name="s",
                                  num_cores=2, num_subcores=16)
    fn = pl.kernel(
        _sc_body,
        compiler_params=pltpu.CompilerParams(needs_layout_passes=False),
        out_type=jax.ShapeDtypeStruct((B, WW), jnp.float32),
        mesh=mesh,
        scratch_types=[
            pltpu.VMEM((WW,), jnp.float32),
            pltpu.VMEM((HH,), jnp.float32),
            pltpu.VMEM((2048,), jnp.float32),
            pltpu.SMEM((128,), jnp.float32),
        ],
    )
    return fn(cond_w, uncond_w)


def _rotl(x, d):
    return lax.shift_left(x, np.int32(d)) | lax.shift_right_logical(x, np.int32(32 - d))


def _threefry(x0, x1):
    ks0 = np.int32(0)
    ks1 = np.int32(SAMPLE_SEED)
    ks2 = np.int32(ks0 ^ ks1 ^ np.int32(0x1BD11BDA))
    rot1 = (13, 15, 26, 6)
    rot2 = (17, 29, 16, 24)
    x0 = x0 + ks0
    x1 = x1 + ks1
    for r in rot1:
        x0 = x0 + x1; x1 = _rotl(x1, r); x1 = x0 ^ x1
    x0 = x0 + ks1; x1 = x1 + ks2 + np.int32(1)
    for r in rot2:
        x0 = x0 + x1; x1 = _rotl(x1, r); x1 = x0 ^ x1
    x0 = x0 + ks2; x1 = x1 + ks0 + np.int32(2)
    for r in rot1:
        x0 = x0 + x1; x1 = _rotl(x1, r); x1 = x0 ^ x1
    x0 = x0 + ks0; x1 = x1 + ks1 + np.int32(3)
    for r in rot2:
        x0 = x0 + x1; x1 = _rotl(x1, r); x1 = x0 ^ x1
    x0 = x0 + ks1; x1 = x1 + ks2 + np.int32(4)
    for r in rot1:
        x0 = x0 + x1; x1 = _rotl(x1, r); x1 = x0 ^ x1
    x0 = x0 + ks2; x1 = x1 + ks0 + np.int32(5)
    return x0, x1


def _gumbel_window():
    row = lax.broadcasted_iota(jnp.int32, (B, WW), 0)
    col = lax.broadcasted_iota(jnp.int32, (B, WW), 1)
    flat = row * np.int32(V) + (col + np.int32(W0))
    o1, o2 = _threefry(jnp.zeros((B, WW), jnp.int32), flat)
    bits = o1 ^ o2
    fb = lax.shift_right_logical(bits, np.int32(9)) | np.int32(0x3F800000)
    f = lax.bitcast_convert_type(fb, jnp.float32) - jnp.float32(1.0)
    tiny = jnp.float32(np.finfo(np.float32).tiny)
    u = jnp.maximum(tiny, f * (jnp.float32(1.0) - tiny) + tiny)
    return -jnp.log(-jnp.log(u))


def _zero_kernel(probs_ref):
    probs_ref[...] = jnp.zeros((B, WW), jnp.float32)


def _tc_kernel(zbuf_ref, cond_ref, uncond_ref, bprobs_ref, probs_ref, ntok_ref,
               band_vmem):
    del zbuf_ref
    col = lax.broadcasted_iota(jnp.int32, (B, WW), 1)
    p2 = bprobs_ref[...]
    keep = p2 > 0.0
    z2 = jnp.sum(p2, axis=1, keepdims=True)
    band_vmem[...] = p2 / z2
    pltpu.sync_copy(band_vmem, probs_ref.at[:, pl.ds(W0, WW)])

    cw = cond_ref[...]
    uw = uncond_ref[...]
    cfg2 = (uw + jnp.float32(CFG_SCALE) * (cw - uw)) / jnp.float32(TEMPERATURE)
    g = _gumbel_window()
    score = jnp.where(keep, cfg2 + g, -jnp.inf)
    smax = jnp.max(score, axis=1, keepdims=True)
    win = jnp.where(score == smax, col, np.int32(2 * WW))
    idx = jnp.min(win, axis=1, keepdims=True) + np.int32(W0)
    ntok_ref[...] = jnp.broadcast_to(idx, (B, 128))


def kernel(cond_logits, uncond_logits):
    cond_w = lax.slice(cond_logits, (0, W0), (B, W0 + WW))
    uncond_w = lax.slice(uncond_logits, (0, W0), (B, W0 + WW))
    zbuf = pl.pallas_call(
        _zero_kernel,
        grid=(N_CHUNK,),
        out_specs=pl.BlockSpec((B, WW), lambda c: (0, c)),
        out_shape=jax.ShapeDtypeStruct((B, V), jnp.float32),
    )()
    bprobs = _sc_topp(cond_w, uncond_w)
    probs, ntok = pl.pallas_call(
        _tc_kernel,
        in_specs=[
            pl.BlockSpec(memory_space=pl.ANY),
            pl.BlockSpec((B, WW), lambda: (0, 0)),
            pl.BlockSpec((B, WW), lambda: (0, 0)),
            pl.BlockSpec((B, WW), lambda: (0, 0)),
        ],
        out_specs=[
            pl.BlockSpec(memory_space=pl.ANY),
            pl.BlockSpec((B, 128), lambda: (0, 0)),
        ],
        out_shape=[
            jax.ShapeDtypeStruct((B, V), jnp.float32),
            jax.ShapeDtypeStruct((B, 128), jnp.int32),
        ],
        scratch_shapes=[pltpu.VMEM((B, WW), jnp.float32)],
        input_output_aliases={0: 0},
    )(zbuf, cond_w, uncond_w, bprobs)
    return probs, ntok[:, 0]
